# baseline (speedup 1.0000x reference)
import jax
import jax.numpy as jnp
from jax import lax
from jax.experimental import pallas as pl
from jax.experimental.pallas import tpu as pltpu
from jax.experimental.pallas import tpu_sc as plsc

NUM_RELS = 1315
NUM_ENTITIES = 1000000
EMB_DIM = 64
BATCH = 16384

NC = 2
NS = 16
NW = NC * NS
L = 16

PER_W = BATCH // NW
CHUNK = 128
NCHUNK = PER_W // CHUNK
NGROUP = CHUNK // L
PITCH = 17


def _rsqrt_nr(x):
    i = lax.bitcast_convert_type(x, jnp.int32)
    z = lax.bitcast_convert_type(
        jnp.int32(0x5F3759DF) - lax.shift_right_arithmetic(i, 1), jnp.float32)
    for _ in range(3):
        z = z * (1.5 - 0.5 * x * z * z)
    return z


def _body(h_hbm, e_hbm, t_hbm, n_hbm, ent_hbm, rel_hbm,
          pos_hbm, neg_hbm,
          hi, ei, ti, ni, hrows, rrows, trows, nrows,
          posv, negv, sem):
    wid = lax.axis_index("s") * NC + lax.axis_index("c")
    base = wid * PER_W

    def chunk_body(ci, carry):
        off = base + ci * CHUNK
        pltpu.sync_copy(h_hbm.at[pl.ds(off, CHUNK)], hi)
        pltpu.sync_copy(e_hbm.at[pl.ds(off, CHUNK)], ei)
        pltpu.sync_copy(t_hbm.at[pl.ds(off, CHUNK)], ti)
        pltpu.sync_copy(n_hbm.at[pl.ds(off, CHUNK)], ni)
        cp_h = pltpu.async_copy(ent_hbm.at[hi], hrows, sem)
        cp_r = pltpu.async_copy(rel_hbm.at[ei], rrows, sem)
        cp_t = pltpu.async_copy(ent_hbm.at[ti], trows, sem)
        cp_n = pltpu.async_copy(ent_hbm.at[ni], nrows, sem)
        cp_h.wait(); cp_r.wait(); cp_t.wait(); cp_n.wait()

        def group_body(g, carry2):
            rbase = g * L
            lane = lax.iota(jnp.int32, L)
            A = jnp.zeros((L,), jnp.float32)
            An = jnp.zeros((L,), jnp.float32)
            B = jnp.zeros((L,), jnp.float32)
            Bn = jnp.zeros((L,), jnp.float32)
            C = jnp.zeros((L,), jnp.float32)
            for r in range(L):
                row = rbase + r
                apos = jnp.zeros((L,), jnp.float32)
                aneg = jnp.zeros((L,), jnp.float32)
                bpos = jnp.zeros((L,), jnp.float32)
                bneg = jnp.zeros((L,), jnp.float32)
                cacc = jnp.zeros((L,), jnp.float32)
                for k in range(EMB_DIM // L):
                    hk = hrows[row, pl.ds(k * L, L)]
                    rk = rrows[row, pl.ds(k * L, L)]
                    tk = trows[row, pl.ds(k * L, L)]
                    nk = nrows[row, pl.ds(k * L, L)]
                    dp = hk - tk
                    dn = hk - nk
                    apos = apos + dp * dp
                    aneg = aneg + dn * dn
                    bpos = bpos + rk * dp
                    bneg = bneg + rk * dn
                    cacc = cacc + rk * rk
                m = lane == r
                A = jnp.where(m, jnp.sum(apos), A)
                An = jnp.where(m, jnp.sum(aneg), An)
                B = jnp.where(m, jnp.sum(bpos), B)
                Bn = jnp.where(m, jnp.sum(bneg), Bn)
                C = jnp.where(m, jnp.sum(cacc), C)

            s = jnp.minimum(_rsqrt_nr(C), 1.0)
            sc = s * C
            psq = jnp.maximum(A + s * (2.0 * B + sc), 0.0)
            nsq = jnp.maximum(An + s * (2.0 * Bn + sc), 0.0)
            obase = ci * CHUNK + rbase
            posv[pl.ds(obase, L)] = psq * _rsqrt_nr(psq)
            negv[pl.ds(obase, L)] = nsq * _rsqrt_nr(nsq)
            return carry2

        return lax.fori_loop(0, NGROUP, group_body, carry)

    lax.fori_loop(0, NCHUNK, chunk_body, 0)

    pltpu.sync_copy(posv, pos_hbm.at[pl.ds(base, PER_W)])
    pltpu.sync_copy(negv, neg_hbm.at[pl.ds(base, PER_W)])


def kernel(h_id, e_id, t_id, neg_id, entity_emb, rel_emb):
    mesh = plsc.VectorSubcoreMesh(core_axis_name="c", subcore_axis_name="s")
    f32 = jnp.float32
    run = pl.kernel(
        _body,
        out_type=(jax.ShapeDtypeStruct((BATCH,), f32),
                  jax.ShapeDtypeStruct((BATCH,), f32)),
        mesh=mesh,
        compiler_params=pltpu.CompilerParams(needs_layout_passes=False,
                                             use_tc_tiling_on_sc=False),
        scratch_types=[
            pltpu.VMEM((CHUNK,), jnp.int32),
            pltpu.VMEM((CHUNK,), jnp.int32),
            pltpu.VMEM((CHUNK,), jnp.int32),
            pltpu.VMEM((CHUNK,), jnp.int32),
            pltpu.VMEM((CHUNK, EMB_DIM), f32),
            pltpu.VMEM((CHUNK, EMB_DIM), f32),
            pltpu.VMEM((CHUNK, EMB_DIM), f32),
            pltpu.VMEM((CHUNK, EMB_DIM), f32),
            pltpu.VMEM((PER_W,), f32),
            pltpu.VMEM((PER_W,), f32),
            pltpu.SemaphoreType.DMA,
        ],
    )
    pos, neg = run(h_id.astype(jnp.int32), e_id.astype(jnp.int32),
                   t_id.astype(jnp.int32), neg_id.astype(jnp.int32),
                   entity_emb, rel_emb)
    return pos, neg
